# baseline (speedup 1.0000x reference)
import functools

import jax
import jax.numpy as jnp
from jax import lax
from jax.experimental import pallas as pl
from jax.experimental.pallas import tpu as pltpu
from jax.experimental.pallas import tpu_sc as plsc

D_MODEL = 64
NUM_CORES = 2
NUM_SUBCORES = 16
NUM_WORKERS = NUM_CORES * NUM_SUBCORES
BATCH = 4096
HIST = 200
NBLK = BATCH // 128
NSLAB = HIST * NBLK
SLAB_PER_W = NSLAB // NUM_WORKERS


def _emb_body(x_hbm, tab_hbm, out_hbm, idx_all, rows0, rows1,
              slabA, slabB, gsem0, gsem1, wsem):
    wid = lax.axis_index("s") * NUM_CORES + lax.axis_index("c")
    gbase = wid * SLAB_PER_W
    pltpu.sync_copy(x_hbm.at[pl.ds(gbase // 2, SLAB_PER_W // 2)], idx_all)
    pltpu.async_copy(tab_hbm.at[idx_all.at[0]], rows0, gsem0)

    lanes = [lax.iota(jnp.int32, 16) + 16 * blk for blk in range(8)]
    jrs = [lanes[c] // 8 for c in range(4)]
    jss = [lanes[c] % 8 for c in range(4)]
    slabs = (slabA, slabB)

    def transpose_half(rows_cur, base, slab_v):
        one = jnp.full((16,), 1, jnp.int32)
        kv = jnp.full((16,), 0, jnp.int32)
        pend = None
        for k in range(128):
            cur = [rows_cur[base + k, pl.ds(16 * c, 16)] for c in range(4)]
            if pend is not None:
                pkv, pvals = pend
                for c in range(4):
                    plsc.store_scatter(slab_v, [jrs[c], jss[c], pkv],
                                       pvals[c])
            pend = (kv, cur)
            kv = kv + one
        pkv, pvals = pend
        for c in range(4):
            plsc.store_scatter(slab_v, [jrs[c], jss[c], pkv], pvals[c])

    def do_pair(p, rows_cur, gsem_cur, rows_nxt, gsem_nxt):
        pltpu.make_async_copy(tab_hbm.at[idx_all.at[p]], rows_cur,
                              gsem_cur).wait()
        p_next = jnp.minimum(p + 1, SLAB_PER_W // 2 - 1)
        pltpu.async_copy(tab_hbm.at[idx_all.at[p_next]], rows_nxt, gsem_nxt)
        for half in (0, 1):
            s = 2 * p + half
            g = gbase + s
            hr = g // (NBLK * 8)
            bc = (g // 8) % NBLK
            hs = g % 8
            h = 8 * hr + hs
            slab_v = slabs[half]

            @pl.when(s > 1)
            def _drain():
                pltpu.make_async_copy(slab_v.at[:, :, pl.ds(0, 128)],
                                      out_hbm.at[h, :, bc], wsem).wait()

            transpose_half(rows_cur, 128 * half, slab_v)
            pltpu.async_copy(slab_v.at[:, :, pl.ds(0, 128)],
                             out_hbm.at[h, :, bc], wsem)

    def step(p2, carry):
        do_pair(2 * p2, rows0, gsem0, rows1, gsem1)
        do_pair(2 * p2 + 1, rows1, gsem1, rows0, gsem0)
        return carry

    lax.fori_loop(0, SLAB_PER_W // 4, step, 0)
    pltpu.make_async_copy(tab_hbm.at[idx_all.at[0]], rows0, gsem0).wait()
    for half in (0, 1):
        pltpu.make_async_copy(slabs[half].at[:, :, pl.ds(0, 128)],
                              out_hbm.at[0, :, 0], wsem).wait()


def kernel(x, table):
    tab2 = table.reshape(table.shape[0] // 2, 2 * D_MODEL)
    tab2 = lax.optimization_barrier(tab2)
    tab_lin = tab2.reshape(table.shape)

    x4 = x.T.reshape(HIST // 8, 8, NBLK, 128).transpose(0, 2, 1, 3)
    x6400 = x4.reshape(NSLAB // 2, 256)

    mesh = plsc.VectorSubcoreMesh(core_axis_name="c", subcore_axis_name="s")
    out5 = pl.kernel(
        _emb_body,
        out_type=jax.ShapeDtypeStruct((HIST, 8, NBLK, 8, 128), jnp.float32),
        mesh=mesh,
        scratch_types=[
            pltpu.VMEM((SLAB_PER_W // 2, 256), jnp.int32),
            pltpu.VMEM((256, D_MODEL), jnp.float32),
            pltpu.VMEM((256, D_MODEL), jnp.float32),
            pltpu.VMEM((8, 8, 129), jnp.float32),
            pltpu.VMEM((8, 8, 129), jnp.float32),
            pltpu.SemaphoreType.DMA,
            pltpu.SemaphoreType.DMA,
            pltpu.SemaphoreType.DMA,
        ],
        compiler_params=pltpu.CompilerParams(
            use_tc_tiling_on_sc=False, needs_layout_passes=False
        ),
    )(x6400, tab_lin)

    out5 = lax.optimization_barrier(out5)
    out = out5.transpose(2, 4, 0, 1, 3).reshape(BATCH, HIST, D_MODEL)
    return out


# speedup vs baseline: 1.0303x; 1.0303x over previous
import functools

import jax
import jax.numpy as jnp
from jax import lax
from jax.experimental import pallas as pl
from jax.experimental.pallas import tpu as pltpu
from jax.experimental.pallas import tpu_sc as plsc

D_MODEL = 64
NUM_CORES = 2
NUM_SUBCORES = 16
NUM_WORKERS = NUM_CORES * NUM_SUBCORES
BATCH = 4096
HIST = 200
NBLK = BATCH // 128
NSLAB = HIST * NBLK
SLAB_PER_W = NSLAB // NUM_WORKERS


def _emb_body(x_hbm, tab_hbm, out_hbm, idx_all, rows0, rows1, slab_v,
              gsem0, gsem1, wsem):
    wid = lax.axis_index("s") * NUM_CORES + lax.axis_index("c")
    gbase = wid * SLAB_PER_W
    pltpu.sync_copy(x_hbm.at[pl.ds(gbase, SLAB_PER_W)], idx_all)
    pltpu.async_copy(tab_hbm.at[idx_all.at[0]], rows0, gsem0)

    lanes = [lax.iota(jnp.int32, 16) + 16 * blk for blk in range(8)]
    jrs = [lanes[c] // 8 for c in range(4)]
    jss = [lanes[c] % 8 for c in range(4)]

    def do_slab(s, rows_cur, gsem_cur, rows_nxt, gsem_nxt):
        g = gbase + s
        hr = g // (NBLK * 8)
        bc = (g // 8) % NBLK
        hs = g % 8
        h = 8 * hr + hs
        pltpu.make_async_copy(tab_hbm.at[idx_all.at[s]], rows_cur,
                              gsem_cur).wait()
        s_next = jnp.minimum(s + 1, SLAB_PER_W - 1)
        pltpu.async_copy(tab_hbm.at[idx_all.at[s_next]], rows_nxt, gsem_nxt)

        @pl.when(s > 0)
        def _drain_prev_writes():
            pltpu.make_async_copy(slab_v.at[:, :, pl.ds(0, 128)],
                                  out_hbm.at[h, :, bc], wsem).wait()

        one = jnp.full((16,), 1, jnp.int32)
        kv = jnp.full((16,), 0, jnp.int32)
        pend = None
        for k in range(128):
            cur = [rows_cur[k, pl.ds(16 * c, 16)] for c in range(4)]
            if pend is not None:
                pkv, pvals = pend
                for c in range(4):
                    plsc.store_scatter(slab_v, [jrs[c], jss[c], pkv], pvals[c])
            pend = (kv, cur)
            kv = kv + one
        pkv, pvals = pend
        for c in range(4):
            plsc.store_scatter(slab_v, [jrs[c], jss[c], pkv], pvals[c])
        pltpu.async_copy(slab_v.at[:, :, pl.ds(0, 128)],
                         out_hbm.at[h, :, bc], wsem)

    def pairstep(s2, carry):
        do_slab(2 * s2, rows0, gsem0, rows1, gsem1)
        do_slab(2 * s2 + 1, rows1, gsem1, rows0, gsem0)
        return carry

    lax.fori_loop(0, SLAB_PER_W // 2, pairstep, 0)
    pltpu.make_async_copy(tab_hbm.at[idx_all.at[0]], rows0, gsem0).wait()
    pltpu.make_async_copy(slab_v.at[:, :, pl.ds(0, 128)],
                          out_hbm.at[0, :, 0], wsem).wait()


def kernel(x, table):
    tab2 = table.reshape(table.shape[0] // 2, 2 * D_MODEL)
    tab2 = lax.optimization_barrier(tab2)
    tab_lin = tab2.reshape(table.shape)

    x4 = x.T.reshape(HIST // 8, 8, NBLK, 128).transpose(0, 2, 1, 3)
    x6400 = x4.reshape(NSLAB, 128)

    mesh = plsc.VectorSubcoreMesh(core_axis_name="c", subcore_axis_name="s")
    out5 = pl.kernel(
        _emb_body,
        out_type=jax.ShapeDtypeStruct((HIST, 8, NBLK, 8, 128), jnp.float32),
        mesh=mesh,
        scratch_types=[
            pltpu.VMEM((SLAB_PER_W, 128), jnp.int32),
            pltpu.VMEM((128, D_MODEL), jnp.float32),
            pltpu.VMEM((128, D_MODEL), jnp.float32),
            pltpu.VMEM((8, 8, 129), jnp.float32),
            pltpu.SemaphoreType.DMA,
            pltpu.SemaphoreType.DMA,
            pltpu.SemaphoreType.DMA,
        ],
        compiler_params=pltpu.CompilerParams(
            use_tc_tiling_on_sc=False, needs_layout_passes=False
        ),
    )(x6400, tab_lin)

    out5 = lax.optimization_barrier(out5)
    out = out5.transpose(2, 4, 0, 1, 3).reshape(BATCH, HIST, D_MODEL)
    return out
